# Initial kernel scaffold; baseline (speedup 1.0000x reference)
import jax
import jax.numpy as jnp
from jax.experimental import pallas as pl


def kernel(X, edge_index, W_self, W_neigh, b_g, W_ih, W_hh, b_ih, b_hh, W_fc, b_fc):
    raise NotImplementedError("write your pallas kernel here")


# speedup vs baseline: 6.9199x; 6.9199x over previous
import functools

import jax
import jax.numpy as jnp
from jax import lax
from jax.experimental import pallas as pl
from jax.experimental.pallas import tpu as pltpu
from jax.experimental.pallas import tpu_sc as plsc

_B, _N, _T, _F, _H, _E, _TOUT = 2, 10000, 12, 128, 64, 320000, 3
_NCH = _B * _T // 2
_NC, _NS = 2, 16
_NW = _NC * _NS
_EPW = _E // _NW
_KW = 128
_WIN = 80
_EPWP = _WIN * _KW
_NPAD = 10112
_RPT = _NPAD // _NS
_NB = 1000


def _prep_body(x_ref, wn_ref, ws_ref, bg_ref, y_ref, s_ref):
    wn = wn_ref[...]
    ws = ws_ref[...]
    bg = bg_ref[...]
    for m in range(6):
        x0 = x_ref[0, :, 2 * m, :]
        x1 = x_ref[0, :, 2 * m + 1, :]
        y_ref[m] = jnp.concatenate([x0 @ wn, x1 @ wn], axis=1)
        s_ref[0, :, 2 * m, :] = x0 @ ws + bg
        s_ref[0, :, 2 * m + 1, :] = x1 @ ws + bg


def _prep(X, W_neigh, W_self, b_g):
    grid = (_B, _N // _NB)
    return pl.pallas_call(
        _prep_body,
        grid=grid,
        in_specs=[
            pl.BlockSpec((1, _NB, _T, _F), lambda b, j: (b, j, 0, 0)),
            pl.BlockSpec((_F, _H), lambda b, j: (0, 0)),
            pl.BlockSpec((_F, _H), lambda b, j: (0, 0)),
            pl.BlockSpec((_H,), lambda b, j: (0,)),
        ],
        out_specs=[
            pl.BlockSpec((6, _NB, 2 * _H), lambda b, j: (b, j, 0)),
            pl.BlockSpec((1, _NB, _T, _H), lambda b, j: (b, j, 0, 0)),
        ],
        out_shape=[
            jax.ShapeDtypeStruct((_NCH, _N, 2 * _H), jnp.float32),
            jax.ShapeDtypeStruct((_B, _N, _T, _H), jnp.float32),
        ],
    )(X, W_neigh, W_self, b_g)


def _sc_body(ytab, srcp, dstp, zf, onesr, aggout, degout,
             accum, sidx, didx, rowsv):
    c = lax.axis_index("c")
    s = lax.axis_index("s")
    base = (c * _NS + s) * _WIN
    myrows = pl.ds(s * _RPT, _RPT)

    pltpu.sync_copy(zf.at[myrows], accum.at[myrows])
    pltpu.sync_copy(onesr, rowsv)
    plsc.subcore_barrier()

    def _deg_grp(g, carry):
        pltpu.sync_copy(dstp.at[pl.ds(base + g * 8, 8)], didx)

        def _deg_win(w, carry2):
            pltpu.sync_copy(rowsv, accum.at[didx.at[w]], add=True)
            return carry2

        lax.fori_loop(0, 8, _deg_win, 0)
        return carry

    lax.fori_loop(0, _WIN // 8, _deg_grp, 0)
    plsc.subcore_barrier()
    pltpu.sync_copy(accum.at[myrows], degout.at[pl.ds(c * _NPAD + s * _RPT, _RPT)])

    for k in range(_NCH):
        pltpu.sync_copy(zf.at[myrows], accum.at[myrows])
        plsc.subcore_barrier()
        tabk = ytab.at[k]

        def _grp(g, carry):
            pltpu.sync_copy(srcp.at[pl.ds(base + g * 8, 8)], sidx)
            pltpu.sync_copy(dstp.at[pl.ds(base + g * 8, 8)], didx)

            def _win(w, carry2):
                pltpu.sync_copy(tabk.at[sidx.at[w]], rowsv)
                pltpu.sync_copy(rowsv, accum.at[didx.at[w]], add=True)
                return carry2

            lax.fori_loop(0, 8, _win, 0)
            return carry

        lax.fori_loop(0, _WIN // 8, _grp, 0)
        plsc.subcore_barrier()
        pltpu.sync_copy(accum.at[myrows],
                        aggout.at[pl.ds((c * _NCH + k) * _NPAD + s * _RPT, _RPT)])


def _sc_segsum(ytab, srcp, dstp, zf, onesr):
    mesh = plsc.VectorSubcoreMesh(core_axis_name="c", subcore_axis_name="s",
                                  num_cores=_NC, num_subcores=_NS)
    return pl.kernel(
        _sc_body,
        out_type=[
            jax.ShapeDtypeStruct((_NC * _NCH * _NPAD, 2 * _H), jnp.float32),
            jax.ShapeDtypeStruct((_NC * _NPAD, 2 * _H), jnp.float32),
        ],
        mesh=mesh,
        scratch_types=[
            pltpu.VMEM_SHARED((_NPAD, 2 * _H), jnp.float32),
            pltpu.VMEM((8, _KW), jnp.int32),
            pltpu.VMEM((8, _KW), jnp.int32),
            pltpu.VMEM((_KW, 2 * _H), jnp.float32),
        ],
    )(ytab, srcp, dstp, zf, onesr)


def _gru_body(s_ref, agg_ref, deg_ref, wih_ref, whh_ref, bih_ref, bhh_ref,
              wfc_ref, bfc_ref, gcn_ref, rnn_ref):
    agg = agg_ref[0] + agg_ref[1]
    deg = deg_ref[0, :, 0:1] + deg_ref[1, :, 0:1]
    inv = 1.0 / jnp.maximum(deg, 1.0)

    xts = []
    for t in range(_T):
        a = agg[t // 2, :, (t % 2) * _H:(t % 2 + 1) * _H]
        xt = jnp.maximum(s_ref[0, :, t, :] + a * inv, 0.0)
        gcn_ref[0, :, t, :] = xt
        xts.append(xt)

    wih = wih_ref[...]
    whh = whh_ref[...]
    bih = bih_ref[...]
    bhh = bhh_ref[...]
    h = jnp.zeros((_NB, _H), jnp.float32)
    for t in range(_T):
        gi = xts[t] @ wih + bih
        gh = h @ whh + bhh
        r = jax.nn.sigmoid(gi[:, :_H] + gh[:, :_H])
        z = jax.nn.sigmoid(gi[:, _H:2 * _H] + gh[:, _H:2 * _H])
        n = jnp.tanh(gi[:, 2 * _H:] + r * gh[:, 2 * _H:])
        h = (1.0 - z) * n + z * h
    rnn_ref[0] = h @ wfc_ref[...] + bfc_ref[...]


def _finish(S, agg4, deg3, W_ih, W_hh, b_ih, b_hh, W_fc, b_fc):
    grid = (_B, _N // _NB)
    return pl.pallas_call(
        _gru_body,
        grid=grid,
        in_specs=[
            pl.BlockSpec((1, _NB, _T, _H), lambda b, j: (b, j, 0, 0)),
            pl.BlockSpec((_NC, 6, _NB, 2 * _H), lambda b, j: (0, b, j, 0)),
            pl.BlockSpec((_NC, _NB, 2 * _H), lambda b, j: (0, j, 0)),
            pl.BlockSpec((_H, 3 * _H), lambda b, j: (0, 0)),
            pl.BlockSpec((_H, 3 * _H), lambda b, j: (0, 0)),
            pl.BlockSpec((3 * _H,), lambda b, j: (0,)),
            pl.BlockSpec((3 * _H,), lambda b, j: (0,)),
            pl.BlockSpec((_H, _TOUT), lambda b, j: (0, 0)),
            pl.BlockSpec((_TOUT,), lambda b, j: (0,)),
        ],
        out_specs=[
            pl.BlockSpec((1, _NB, _T, _H), lambda b, j: (b, j, 0, 0)),
            pl.BlockSpec((1, _NB, _TOUT), lambda b, j: (b, j, 0)),
        ],
        out_shape=[
            jax.ShapeDtypeStruct((_B, _N, _T, _H), jnp.float32),
            jax.ShapeDtypeStruct((_B, _N, _TOUT), jnp.float32),
        ],
    )(S, agg4, deg3, W_ih, W_hh, b_ih, b_hh, W_fc, b_fc)


def kernel(X, edge_index, W_self, W_neigh, b_g, W_ih, W_hh, b_ih, b_hh, W_fc, b_fc):
    ytab, S = _prep(X, W_neigh, W_self, b_g)

    src = edge_index[0].reshape(_NW, _EPW)
    dst = edge_index[1].reshape(_NW, _EPW)
    npad = _EPWP - _EPW
    srcp = jnp.concatenate(
        [src, jnp.zeros((_NW, npad), jnp.int32)], axis=1).reshape(_NW * _WIN, _KW)
    trash = _N + (jnp.arange(_NW, dtype=jnp.int32) % (_NPAD - _N))[:, None]
    dstp = jnp.concatenate(
        [dst, jnp.broadcast_to(trash, (_NW, npad))], axis=1).reshape(_NW * _WIN, _KW)

    zf = jnp.zeros((_NPAD, 2 * _H), jnp.float32)
    onesr = jnp.ones((_KW, 2 * _H), jnp.float32)

    aggout, degout = _sc_segsum(ytab, srcp, dstp, zf, onesr)
    agg4 = aggout.reshape(_NC, _NCH, _NPAD, 2 * _H)
    deg3 = degout.reshape(_NC, _NPAD, 2 * _H)

    gcn_out, rnn_out = _finish(S, agg4, deg3, W_ih, W_hh, b_ih, b_hh, W_fc, b_fc)
    return rnn_out, gcn_out


# speedup vs baseline: 7.4857x; 1.0818x over previous
import functools

import jax
import jax.numpy as jnp
from jax import lax
from jax.experimental import pallas as pl
from jax.experimental.pallas import tpu as pltpu
from jax.experimental.pallas import tpu_sc as plsc

_B, _N, _T, _F, _H, _E, _TOUT = 2, 10000, 12, 128, 64, 320000, 3
_NCH = _B * _T // 2
_NC, _NS = 2, 16
_NW = _NC * _NS
_EPW = _E // _NW
_KW = 128
_WIN = 80
_EPWP = _WIN * _KW
_NPAD = 10112
_RPT = _NPAD // _NS
_NB = 1000


def _prep_body(x_ref, wn_ref, ws_ref, bg_ref, y_ref, s_ref):
    wn = wn_ref[...]
    ws = ws_ref[...]
    bg = bg_ref[...]
    for m in range(6):
        x0 = x_ref[0, :, 2 * m, :]
        x1 = x_ref[0, :, 2 * m + 1, :]
        y_ref[m] = jnp.concatenate([x0 @ wn, x1 @ wn], axis=1)
        s_ref[0, :, 2 * m, :] = x0 @ ws + bg
        s_ref[0, :, 2 * m + 1, :] = x1 @ ws + bg


def _prep(X, W_neigh, W_self, b_g):
    grid = (_B, _N // _NB)
    return pl.pallas_call(
        _prep_body,
        grid=grid,
        in_specs=[
            pl.BlockSpec((1, _NB, _T, _F), lambda b, j: (b, j, 0, 0)),
            pl.BlockSpec((_F, _H), lambda b, j: (0, 0)),
            pl.BlockSpec((_F, _H), lambda b, j: (0, 0)),
            pl.BlockSpec((_H,), lambda b, j: (0,)),
        ],
        out_specs=[
            pl.BlockSpec((6, _NB, 2 * _H), lambda b, j: (b, j, 0)),
            pl.BlockSpec((1, _NB, _T, _H), lambda b, j: (b, j, 0, 0)),
        ],
        out_shape=[
            jax.ShapeDtypeStruct((_NCH, _N, 2 * _H), jnp.float32),
            jax.ShapeDtypeStruct((_B, _N, _T, _H), jnp.float32),
        ],
    )(X, W_neigh, W_self, b_g)


def _sc_body(ytab, srcp, dstp, zf, onesr, aggout, degout,
             accum, sidx, didx, r0, r1, gsem, ssem0, ssem1):
    c = lax.axis_index("c")
    s = lax.axis_index("s")
    base = (c * _NS + s) * _WIN
    myrows = pl.ds(s * _RPT, _RPT)
    ngrp = _WIN // 8

    pltpu.sync_copy(zf.at[myrows], accum.at[myrows])
    pltpu.sync_copy(onesr, r0)
    plsc.subcore_barrier()

    def _deg_grp(g, carry):
        pltpu.sync_copy(dstp.at[pl.ds(base + g * 8, 8)], didx)
        for j in range(8):
            pltpu.make_async_copy(r0, accum.at[didx.at[j]], ssem0).start(add=True)
        for j in range(8):
            pltpu.make_async_copy(r0, accum.at[didx.at[j]], ssem0).wait()
        return carry

    lax.fori_loop(0, ngrp, _deg_grp, 0)
    plsc.subcore_barrier()
    pltpu.sync_copy(accum.at[myrows], degout.at[pl.ds(c * _NPAD + s * _RPT, _RPT)])

    bufs = (r0, r1)
    ssems = (ssem0, ssem1)
    for k in range(_NCH):
        pltpu.sync_copy(zf.at[myrows], accum.at[myrows])
        plsc.subcore_barrier()
        tabk = ytab.at[k]

        def _grp(g, carry):
            pltpu.sync_copy(srcp.at[pl.ds(base + g * 8, 8)], sidx)
            pltpu.sync_copy(dstp.at[pl.ds(base + g * 8, 8)], didx)
            pltpu.make_async_copy(tabk.at[sidx.at[0]], r0, gsem).start()
            for j in range(8):
                b = bufs[j % 2]
                ss = ssems[j % 2]
                pltpu.make_async_copy(tabk.at[sidx.at[j]], b, gsem).wait()
                pltpu.make_async_copy(b, accum.at[didx.at[j]], ss).start(add=True)
                if j >= 1:
                    pltpu.make_async_copy(bufs[(j - 1) % 2],
                                          accum.at[didx.at[j - 1]],
                                          ssems[(j - 1) % 2]).wait()
                if j + 1 < 8:
                    pltpu.make_async_copy(tabk.at[sidx.at[j + 1]],
                                          bufs[(j + 1) % 2], gsem).start()
            pltpu.make_async_copy(r1, accum.at[didx.at[7]], ssem1).wait()
            return carry

        lax.fori_loop(0, ngrp, _grp, 0)
        plsc.subcore_barrier()
        pltpu.sync_copy(accum.at[myrows],
                        aggout.at[pl.ds((c * _NCH + k) * _NPAD + s * _RPT, _RPT)])


def _sc_segsum(ytab, srcp, dstp, zf, onesr):
    mesh = plsc.VectorSubcoreMesh(core_axis_name="c", subcore_axis_name="s",
                                  num_cores=_NC, num_subcores=_NS)
    return pl.kernel(
        _sc_body,
        out_type=[
            jax.ShapeDtypeStruct((_NC * _NCH * _NPAD, 2 * _H), jnp.float32),
            jax.ShapeDtypeStruct((_NC * _NPAD, 2 * _H), jnp.float32),
        ],
        mesh=mesh,
        scratch_types=[
            pltpu.VMEM_SHARED((_NPAD, 2 * _H), jnp.float32),
            pltpu.VMEM((8, _KW), jnp.int32),
            pltpu.VMEM((8, _KW), jnp.int32),
            pltpu.VMEM((_KW, 2 * _H), jnp.float32),
            pltpu.VMEM((_KW, 2 * _H), jnp.float32),
            pltpu.SemaphoreType.DMA,
            pltpu.SemaphoreType.DMA,
            pltpu.SemaphoreType.DMA,
        ],
    )(ytab, srcp, dstp, zf, onesr)


def _gru_body(s_ref, agg_ref, deg_ref, wih_ref, whh_ref, bih_ref, bhh_ref,
              wfc_ref, bfc_ref, gcn_ref, rnn_ref):
    agg = agg_ref[0] + agg_ref[1]
    deg = deg_ref[0, :, 0:1] + deg_ref[1, :, 0:1]
    inv = 1.0 / jnp.maximum(deg, 1.0)

    xts = []
    for t in range(_T):
        a = agg[t // 2, :, (t % 2) * _H:(t % 2 + 1) * _H]
        xt = jnp.maximum(s_ref[0, :, t, :] + a * inv, 0.0)
        gcn_ref[0, :, t, :] = xt
        xts.append(xt)

    wih = wih_ref[...]
    whh = whh_ref[...]
    bih = bih_ref[...]
    bhh = bhh_ref[...]
    h = jnp.zeros((_NB, _H), jnp.float32)
    for t in range(_T):
        gi = xts[t] @ wih + bih
        gh = h @ whh + bhh
        r = jax.nn.sigmoid(gi[:, :_H] + gh[:, :_H])
        z = jax.nn.sigmoid(gi[:, _H:2 * _H] + gh[:, _H:2 * _H])
        n = jnp.tanh(gi[:, 2 * _H:] + r * gh[:, 2 * _H:])
        h = (1.0 - z) * n + z * h
    rnn_ref[0] = h @ wfc_ref[...] + bfc_ref[...]


def _finish(S, agg4, deg3, W_ih, W_hh, b_ih, b_hh, W_fc, b_fc):
    grid = (_B, _N // _NB)
    return pl.pallas_call(
        _gru_body,
        grid=grid,
        in_specs=[
            pl.BlockSpec((1, _NB, _T, _H), lambda b, j: (b, j, 0, 0)),
            pl.BlockSpec((_NC, 6, _NB, 2 * _H), lambda b, j: (0, b, j, 0)),
            pl.BlockSpec((_NC, _NB, 2 * _H), lambda b, j: (0, j, 0)),
            pl.BlockSpec((_H, 3 * _H), lambda b, j: (0, 0)),
            pl.BlockSpec((_H, 3 * _H), lambda b, j: (0, 0)),
            pl.BlockSpec((3 * _H,), lambda b, j: (0,)),
            pl.BlockSpec((3 * _H,), lambda b, j: (0,)),
            pl.BlockSpec((_H, _TOUT), lambda b, j: (0, 0)),
            pl.BlockSpec((_TOUT,), lambda b, j: (0,)),
        ],
        out_specs=[
            pl.BlockSpec((1, _NB, _T, _H), lambda b, j: (b, j, 0, 0)),
            pl.BlockSpec((1, _NB, _TOUT), lambda b, j: (b, j, 0)),
        ],
        out_shape=[
            jax.ShapeDtypeStruct((_B, _N, _T, _H), jnp.float32),
            jax.ShapeDtypeStruct((_B, _N, _TOUT), jnp.float32),
        ],
    )(S, agg4, deg3, W_ih, W_hh, b_ih, b_hh, W_fc, b_fc)


def kernel(X, edge_index, W_self, W_neigh, b_g, W_ih, W_hh, b_ih, b_hh, W_fc, b_fc):
    ytab, S = _prep(X, W_neigh, W_self, b_g)

    src = edge_index[0].reshape(_NW, _EPW)
    dst = edge_index[1].reshape(_NW, _EPW)
    npad = _EPWP - _EPW
    srcp = jnp.concatenate(
        [src, jnp.zeros((_NW, npad), jnp.int32)], axis=1).reshape(_NW * _WIN, _KW)
    trash = _N + (jnp.arange(_NW, dtype=jnp.int32) % (_NPAD - _N))[:, None]
    dstp = jnp.concatenate(
        [dst, jnp.broadcast_to(trash, (_NW, npad))], axis=1).reshape(_NW * _WIN, _KW)

    zf = jnp.zeros((_NPAD, 2 * _H), jnp.float32)
    onesr = jnp.ones((_KW, 2 * _H), jnp.float32)

    aggout, degout = _sc_segsum(ytab, srcp, dstp, zf, onesr)
    agg4 = aggout.reshape(_NC, _NCH, _NPAD, 2 * _H)
    deg3 = degout.reshape(_NC, _NPAD, 2 * _H)

    gcn_out, rnn_out = _finish(S, agg4, deg3, W_ih, W_hh, b_ih, b_hh, W_fc, b_fc)
    return rnn_out, gcn_out
